# baseline (speedup 1.0000x reference)
import functools

import jax
import jax.numpy as jnp
import numpy as np
from jax import lax
from jax.experimental import pallas as pl
from jax.experimental.pallas import tpu as pltpu
from jax.experimental.pallas import tpu_sc as plsc

BATCH = 4096
SEQ = 200
D = 64
VOCAB = 1000000
L = 16
NC = 2
NS = 16
NW = NC * NS
B_PER_W = BATCH // NW
TOK_PER_W = B_PER_W * SEQ
SPLIT = 128
REST = SEQ - SPLIT

BLK = 65536
NBLK = -(-VOCAB // BLK)
VOCAB_PAD = NBLK * BLK

PERM = np.concatenate(
    [np.arange(0, 16), np.arange(32, 48),
     np.arange(16, 32), np.arange(48, 64)]
)


def _tc_transpose(table_t):
    Q = BLK // 4

    def rne(f):
        b = jax.lax.bitcast_convert_type(f, jnp.int32)
        return (b + 0x7FFF + ((b >> 16) & 1)) >> 16

    def body(x_ref, o_ref):
        x = x_ref[...]
        xl = jnp.concatenate(
            [x[0:32, i * Q : (i + 1) * Q] for i in range(4)], axis=0
        )
        xh = jnp.concatenate(
            [x[32:64, i * Q : (i + 1) * Q] for i in range(4)], axis=0
        )
        o_ref[...] = (rne(xl.T) & 0xFFFF) | (rne(xh.T) << 16)

    return pl.pallas_call(
        body,
        grid=(NBLK,),
        in_specs=[pl.BlockSpec((D, BLK), lambda i: (0, i))],
        out_shape=jax.ShapeDtypeStruct((VOCAB_PAD // 4, 2 * D), jnp.int32),
        out_specs=pl.BlockSpec((Q, 2 * D), lambda i: (i, 0)),
    )(table_t)


def _sc_gather_mean(token_ids_flat, table_lin):
    mesh = plsc.VectorSubcoreMesh(
        core_axis_name="c", subcore_axis_name="s", num_cores=NC, num_subcores=NS
    )
    NBUF = 8

    @functools.partial(
        pl.kernel,
        out_type=jax.ShapeDtypeStruct((BATCH, D), jnp.float32),
        mesh=mesh,
        compiler_params=pltpu.CompilerParams(
            use_tc_tiling_on_sc=False, needs_layout_passes=False
        ),
        scratch_types=[
            pltpu.VMEM((TOK_PER_W,), jnp.int32),
            [pltpu.VMEM((SEQ, D // 2), jnp.int32) for _ in range(NBUF)],
            pltpu.VMEM((B_PER_W, D), jnp.float32),
            [pltpu.SemaphoreType.DMA for _ in range(NBUF)],
        ],
    )
    def k(tok_hbm, table_hbm, out_hbm, idx_v, rows, out_v, sems):
        wid = lax.axis_index("s") * NC + lax.axis_index("c")
        base = wid * B_PER_W
        pltpu.sync_copy(tok_hbm.at[pl.ds(wid * TOK_PER_W, TOK_PER_W)], idx_v)

        sh = BLK.bit_length() - 3

        @plsc.parallel_loop(0, TOK_PER_W // L, unroll=8)
        def _remap(c):
            v = idx_v[pl.ds(c * L, L)]
            idx_v[pl.ds(c * L, L)] = (
                (v & ~(BLK - 1))
                + ((v & (BLK // 4 - 1)) << 2)
                + ((v & (BLK - 1)) >> sh)
            )

        def start(i, buf, sem):
            pltpu.async_copy(
                table_hbm.at[idx_v.at[pl.ds(i * SEQ, SPLIT)]],
                buf.at[pl.ds(0, SPLIT)],
                sem,
            )
            pltpu.async_copy(
                table_hbm.at[idx_v.at[pl.ds(i * SEQ + SPLIT, REST)]],
                buf.at[pl.ds(SPLIT, REST)],
                sem,
            )

        def drain(buf, sem):
            pltpu.make_async_copy(table_hbm.at[pl.ds(0, SEQ)], buf, sem).wait()

        for b in range(NBUF):
            start(b, rows[b], sems[b])

        mask_hi = jnp.full((L,), -65536, jnp.int32)

        def group_body(g, carry):
            i0 = g * NBUF
            for b in range(NBUF):
                i = i0 + b
                drain(rows[b], sems[b])
                buf = rows[b]

                @plsc.parallel_loop(
                    0,
                    SEQ,
                    unroll=8,
                    carry=tuple(
                        jnp.zeros((L,), jnp.float32) for _ in range(4)
                    ),
                )
                def acc(j, c):
                    out = []
                    for h in range(2):
                        v = buf[j, pl.ds(h * L, L)]
                        even = plsc.bitcast(v << 16, jnp.float32)
                        odd = plsc.bitcast(v & mask_hi, jnp.float32)
                        out.append(c[2 * h] + even)
                        out.append(c[2 * h + 1] + odd)
                    return tuple(out)

                scale = jnp.float32(1.0 / SEQ)
                for d in range(4):
                    out_v[i, pl.ds(d * L, L)] = acc[d] * scale

                @pl.when(g < B_PER_W // NBUF - 1)
                def _():
                    start(i + NBUF, rows[b], sems[b])

            return carry

        lax.fori_loop(0, B_PER_W // NBUF, group_body, 0)
        pltpu.sync_copy(out_v, out_hbm.at[pl.ds(base, B_PER_W)])

    return k(token_ids_flat, table_lin)


def _tc_linear(x, wt, b2):
    def mm(x_ref, w_ref, b_ref, o_ref):
        o_ref[...] = (
            jnp.dot(x_ref[...], w_ref[...], preferred_element_type=jnp.float32)
            + b_ref[...]
        )

    return pl.pallas_call(
        mm,
        out_shape=jax.ShapeDtypeStruct((BATCH, D), jnp.float32),
    )(x, wt, b2)


def kernel(token_ids, emb_table, W, b):
    table_lin = _tc_transpose(emb_table.T).reshape(VOCAB_PAD, D // 2)
    tok_flat = token_ids.astype(jnp.int32).reshape(BATCH * SEQ)
    x = _sc_gather_mean(tok_flat, table_lin)
    return _tc_linear(x, W.T[PERM, :], b.reshape(1, D))


# speedup vs baseline: 1.0074x; 1.0074x over previous
import functools

import jax
import jax.numpy as jnp
import numpy as np
from jax import lax
from jax.experimental import pallas as pl
from jax.experimental.pallas import tpu as pltpu
from jax.experimental.pallas import tpu_sc as plsc

BATCH = 4096
SEQ = 200
D = 64
VOCAB = 1000000
L = 16
NC = 2
NS = 16
NW = NC * NS
B_PER_W = BATCH // NW
TOK_PER_W = B_PER_W * SEQ
SPLIT = 128
REST = SEQ - SPLIT

BLK = 32768
NBLK = -(-VOCAB // BLK)
VOCAB_PAD = NBLK * BLK

PERM = np.concatenate(
    [np.arange(0, 16), np.arange(32, 48),
     np.arange(16, 32), np.arange(48, 64)]
)


def _tc_transpose(table_t):
    Q = BLK // 4

    def rne(f):
        b = jax.lax.bitcast_convert_type(f, jnp.int32)
        return (b + 0x7FFF + ((b >> 16) & 1)) >> 16

    def body(x_ref, o_ref):
        x = x_ref[...]
        xl = jnp.concatenate(
            [x[0:32, i * Q : (i + 1) * Q] for i in range(4)], axis=0
        )
        xh = jnp.concatenate(
            [x[32:64, i * Q : (i + 1) * Q] for i in range(4)], axis=0
        )
        o_ref[...] = (rne(xl.T) & 0xFFFF) | (rne(xh.T) << 16)

    return pl.pallas_call(
        body,
        grid=(NBLK,),
        in_specs=[pl.BlockSpec((D, BLK), lambda i: (0, i))],
        out_shape=jax.ShapeDtypeStruct((VOCAB_PAD // 4, 2 * D), jnp.int32),
        out_specs=pl.BlockSpec((Q, 2 * D), lambda i: (i, 0)),
    )(table_t)


def _sc_gather_mean(token_ids_flat, table_lin):
    mesh = plsc.VectorSubcoreMesh(
        core_axis_name="c", subcore_axis_name="s", num_cores=NC, num_subcores=NS
    )
    NBUF = 8

    @functools.partial(
        pl.kernel,
        out_type=jax.ShapeDtypeStruct((BATCH, D), jnp.float32),
        mesh=mesh,
        compiler_params=pltpu.CompilerParams(
            use_tc_tiling_on_sc=False, needs_layout_passes=False
        ),
        scratch_types=[
            pltpu.VMEM((TOK_PER_W,), jnp.int32),
            [pltpu.VMEM((SEQ, D // 2), jnp.int32) for _ in range(NBUF)],
            pltpu.VMEM((B_PER_W, D), jnp.float32),
            [pltpu.SemaphoreType.DMA for _ in range(NBUF)],
        ],
    )
    def k(tok_hbm, table_hbm, out_hbm, idx_v, rows, out_v, sems):
        wid = lax.axis_index("s") * NC + lax.axis_index("c")
        base = wid * B_PER_W
        pltpu.sync_copy(tok_hbm.at[pl.ds(wid * TOK_PER_W, TOK_PER_W)], idx_v)

        sh = BLK.bit_length() - 3

        @plsc.parallel_loop(0, TOK_PER_W // L, unroll=8)
        def _remap(c):
            v = idx_v[pl.ds(c * L, L)]
            idx_v[pl.ds(c * L, L)] = (
                (v & ~(BLK - 1))
                + ((v & (BLK // 4 - 1)) << 2)
                + ((v & (BLK - 1)) >> sh)
            )

        def start(i, buf, sem):
            pltpu.async_copy(
                table_hbm.at[idx_v.at[pl.ds(i * SEQ, SPLIT)]],
                buf.at[pl.ds(0, SPLIT)],
                sem,
            )
            pltpu.async_copy(
                table_hbm.at[idx_v.at[pl.ds(i * SEQ + SPLIT, REST)]],
                buf.at[pl.ds(SPLIT, REST)],
                sem,
            )

        def drain(buf, sem):
            pltpu.make_async_copy(table_hbm.at[pl.ds(0, SEQ)], buf, sem).wait()

        for b in range(NBUF):
            start(b, rows[b], sems[b])

        mask_hi = jnp.full((L,), -65536, jnp.int32)

        def group_body(g, carry):
            i0 = g * NBUF
            for b in range(NBUF):
                i = i0 + b
                drain(rows[b], sems[b])
                buf = rows[b]

                @plsc.parallel_loop(
                    0,
                    SEQ,
                    unroll=8,
                    carry=tuple(
                        jnp.zeros((L,), jnp.float32) for _ in range(4)
                    ),
                )
                def acc(j, c):
                    out = []
                    for h in range(2):
                        v = buf[j, pl.ds(h * L, L)]
                        even = plsc.bitcast(v << 16, jnp.float32)
                        odd = plsc.bitcast(v & mask_hi, jnp.float32)
                        out.append(c[2 * h] + even)
                        out.append(c[2 * h + 1] + odd)
                    return tuple(out)

                scale = jnp.float32(1.0 / SEQ)
                for d in range(4):
                    out_v[i, pl.ds(d * L, L)] = acc[d] * scale

                @pl.when(g < B_PER_W // NBUF - 1)
                def _():
                    start(i + NBUF, rows[b], sems[b])

            return carry

        lax.fori_loop(0, B_PER_W // NBUF, group_body, 0)
        pltpu.sync_copy(out_v, out_hbm.at[pl.ds(base, B_PER_W)])

    return k(token_ids_flat, table_lin)


def _tc_linear(x, wt, b2):
    def mm(x_ref, w_ref, b_ref, o_ref):
        o_ref[...] = (
            jnp.dot(x_ref[...], w_ref[...], preferred_element_type=jnp.float32)
            + b_ref[...]
        )

    return pl.pallas_call(
        mm,
        out_shape=jax.ShapeDtypeStruct((BATCH, D), jnp.float32),
    )(x, wt, b2)


def kernel(token_ids, emb_table, W, b):
    table_lin = _tc_transpose(emb_table.T).reshape(VOCAB_PAD, D // 2)
    tok_flat = token_ids.astype(jnp.int32).reshape(BATCH * SEQ)
    x = _sc_gather_mean(tok_flat, table_lin)
    return _tc_linear(x, W.T[PERM, :], b.reshape(1, D))
